# baseline (speedup 1.0000x reference)
import functools

import jax
import jax.numpy as jnp
from jax import lax
from jax.experimental import pallas as pl
from jax.experimental.pallas import tpu as pltpu
from jax.experimental.pallas import tpu_sc as plsc

IN_LEN = 16384
OUT_LEN = 1024
NNZ = 262144
BATCH = 16
LANES = 16

NUM_CORES = 2
NUM_SUBCORES = 16
NW = NUM_CORES * NUM_SUBCORES
PER_W = NNZ // NW
SEG = 128
CHUNK = 1024
NSEG = CHUNK // SEG
NCHUNK = PER_W // CHUNK
GROUPS = CHUNK // LANES
NBUF = 4


def _sc_body(xt_hbm, rows_hbm, cols_hbm, w_hbm, out_hbm,
             rows_v, cols_v, w_v, gath_v, acc_sh, sem_idx, sem_gat, sem_sca):
    cid = lax.axis_index("c")
    sid = lax.axis_index("s")
    wid = sid * NUM_CORES + cid
    iota_l = lax.iota(jnp.int32, LANES)

    zrows = OUT_LEN // NUM_SUBCORES
    zero = jnp.zeros((LANES,), jnp.float32)

    @plsc.parallel_loop(0, zrows, unroll=4)
    def zb(i):
        gath_v[0, i, :] = zero

    pltpu.sync_copy(gath_v.at[0, pl.ds(0, zrows)],
                    acc_sh.at[pl.ds(sid * zrows, zrows)])
    plsc.subcore_barrier()

    def issue_idx(t):
        b = t % NBUF
        nnz_base = pl.multiple_of(wid * PER_W + t * CHUNK, CHUNK)
        seg_base = pl.multiple_of(nnz_base // SEG, NSEG)
        return [
            pltpu.async_copy(rows_hbm.at[pl.ds(seg_base, NSEG)],
                             rows_v.at[b], sem_idx.at[b]),
            pltpu.async_copy(cols_hbm.at[pl.ds(seg_base, NSEG)],
                             cols_v.at[b], sem_idx.at[b]),
            pltpu.async_copy(w_hbm.at[pl.ds(nnz_base, CHUNK)],
                             w_v.at[b], sem_idx.at[b]),
        ]

    def issue_gat(t):
        b = t % NBUF
        return [
            pltpu.async_copy(xt_hbm.at[rows_v.at[b, s]],
                             gath_v.at[b, pl.ds(s * SEG, SEG)], sem_gat.at[b])
            for s in range(NSEG)
        ]

    def issue_sca(t):
        b = t % NBUF
        return [
            pltpu.async_copy(gath_v.at[b, pl.ds(s * SEG, SEG)],
                             acc_sh.at[cols_v.at[b, s]], sem_sca.at[b], add=True)
            for s in range(NSEG)
        ]

    def scale(t):
        b = t % NBUF

        @plsc.parallel_loop(0, GROUPS, unroll=4)
        def grp(g):
            gb = g * LANES
            for j in range(LANES):
                pos = gb + j
                wj = plsc.load_gather(
                    w_v.at[b], [jnp.full((LANES,), pos, jnp.int32)])
                gath_v[b, pos, :] = wj * gath_v[b, pos, :]

    idx_d = {0: issue_idx(0), 1: issue_idx(1)}
    for d in idx_d[0]:
        d.wait()
    gat_d = {0: issue_gat(0)}
    sca_d = {}
    for t in range(NCHUNK):
        if t >= 2:
            for d in sca_d[t - 2]:
                d.wait()
        if t + 2 < NCHUNK:
            idx_d[t + 2] = issue_idx(t + 2)
        if t + 1 < NCHUNK:
            for d in idx_d[t + 1]:
                d.wait()
            gat_d[t + 1] = issue_gat(t + 1)
        for d in gat_d[t]:
            d.wait()
        scale(t)
        sca_d[t] = issue_sca(t)
    for t in range(NCHUNK - 2, NCHUNK):
        for d in sca_d[t]:
            d.wait()

    plsc.subcore_barrier()

    pltpu.sync_copy(acc_sh.at[pl.ds(sid * zrows, zrows)],
                    out_hbm.at[cid, pl.ds(sid * zrows, zrows)])


_sc_call = pl.kernel(
    _sc_body,
    out_type=jax.ShapeDtypeStruct((NUM_CORES, OUT_LEN, BATCH), jnp.float32),
    mesh=plsc.VectorSubcoreMesh(core_axis_name="c", subcore_axis_name="s"),
    compiler_params=pltpu.CompilerParams(needs_layout_passes=False,
                                         use_tc_tiling_on_sc=False),
    scratch_types=[
        pltpu.VMEM((NBUF, NSEG, SEG), jnp.int32),
        pltpu.VMEM((NBUF, NSEG, SEG), jnp.int32),
        pltpu.VMEM((NBUF, CHUNK), jnp.float32),
        pltpu.VMEM((NBUF, CHUNK, LANES), jnp.float32),
        pltpu.VMEM_SHARED((OUT_LEN, BATCH), jnp.float32),
        pltpu.SemaphoreType.DMA((NBUF,)),
        pltpu.SemaphoreType.DMA((NBUF,)),
        pltpu.SemaphoreType.DMA((NBUF,)),
    ],
)


def _combine_body(parts_ref, bias_ref, out_ref):
    p = (parts_ref[0:OUT_LEN, :] + parts_ref[OUT_LEN:, :]) + bias_ref[...]
    out_ref[...] = p.T


_combine_call = pl.pallas_call(
    _combine_body,
    out_shape=jax.ShapeDtypeStruct((BATCH, OUT_LEN), jnp.float32),
)


def kernel(inputs, kernel, bias, mask_rows, mask_cols):
    xt = inputs[:, :, 0].T
    w = kernel[:, 0]
    rows2d = mask_rows.reshape(NNZ // SEG, SEG)
    cols2d = mask_cols.reshape(NNZ // SEG, SEG)
    parts = _sc_call(xt, rows2d, cols2d, w)
    out_t = _combine_call(parts.reshape(NUM_CORES * OUT_LEN, BATCH), bias)
    return out_t.reshape(BATCH, OUT_LEN, 1)


# speedup vs baseline: 1.0636x; 1.0636x over previous
import functools

import jax
import jax.numpy as jnp
from jax import lax
from jax.experimental import pallas as pl
from jax.experimental.pallas import tpu as pltpu
from jax.experimental.pallas import tpu_sc as plsc

IN_LEN = 16384
OUT_LEN = 1024
NNZ = 262144
BATCH = 16
LANES = 16

NUM_CORES = 2
NUM_SUBCORES = 16
NW = NUM_CORES * NUM_SUBCORES
PER_W = NNZ // NW
SEG = 128
CHUNK = 1024
NSEG = CHUNK // SEG
NCHUNK = PER_W // CHUNK
GROUPS = CHUNK // LANES
NBUF = 4


XPAD = 1025


def _sc_body(x_hbm, rows_hbm, cols_hbm, w_hbm, out_hbm, xt_s,
             rows_v, cols_v, w_v, gath_v, xb_v, acc_sh,
             sem_idx, sem_gat, sem_sca, sem_tr):
    cid = lax.axis_index("c")
    sid = lax.axis_index("s")
    wid = sid * NUM_CORES + cid
    iota_l = lax.iota(jnp.int32, LANES)

    tcols = IN_LEN // NUM_SUBCORES
    col0 = pl.multiple_of(sid * tcols, tcols)
    tds = [
        pltpu.async_copy(x_hbm.at[b, pl.ds(col0, tcols)],
                         xb_v.at[b, pl.ds(0, tcols)], sem_tr)
        for b in range(BATCH)
    ]
    for d in tds:
        d.wait()
    @plsc.parallel_loop(0, tcols, unroll=4)
    def trow(r):
        gath_v[0, r, :] = plsc.load_gather(
            xb_v, [iota_l, jnp.full((LANES,), r, jnp.int32)])

    pltpu.async_copy(gath_v.at[0], xt_s.at[cid, pl.ds(col0, tcols)],
                     sem_tr).wait()
    xt_hbm = xt_s.at[cid]

    zrows = OUT_LEN // NUM_SUBCORES
    zero = jnp.zeros((LANES,), jnp.float32)

    @plsc.parallel_loop(0, zrows, unroll=4)
    def zb(i):
        gath_v[1, i, :] = zero

    pltpu.sync_copy(gath_v.at[1, pl.ds(0, zrows)],
                    acc_sh.at[pl.ds(sid * zrows, zrows)])
    plsc.subcore_barrier()

    def issue_idx(t):
        b = t % NBUF
        nnz_base = pl.multiple_of(wid * PER_W + t * CHUNK, CHUNK)
        seg_base = pl.multiple_of(nnz_base // SEG, NSEG)
        return [
            pltpu.async_copy(rows_hbm.at[pl.ds(seg_base, NSEG)],
                             rows_v.at[b], sem_idx.at[b]),
            pltpu.async_copy(cols_hbm.at[pl.ds(seg_base, NSEG)],
                             cols_v.at[b], sem_idx.at[b]),
            pltpu.async_copy(w_hbm.at[pl.ds(nnz_base, CHUNK)],
                             w_v.at[b], sem_idx.at[b]),
        ]

    def issue_gat(t):
        b = t % NBUF
        return [
            pltpu.async_copy(xt_hbm.at[rows_v.at[b, s]],
                             gath_v.at[b, pl.ds(s * SEG, SEG)], sem_gat.at[b])
            for s in range(NSEG)
        ]

    def issue_sca(t):
        b = t % NBUF
        return [
            pltpu.async_copy(gath_v.at[b, pl.ds(s * SEG, SEG)],
                             acc_sh.at[cols_v.at[b, s]], sem_sca.at[b], add=True)
            for s in range(NSEG)
        ]

    def scale(t):
        b = t % NBUF

        @plsc.parallel_loop(0, GROUPS, unroll=4)
        def grp(g):
            gb = g * LANES
            for j in range(LANES):
                pos = gb + j
                wj = plsc.load_gather(
                    w_v.at[b], [jnp.full((LANES,), pos, jnp.int32)])
                gath_v[b, pos, :] = wj * gath_v[b, pos, :]

    idx_d = {0: issue_idx(0), 1: issue_idx(1)}
    for d in idx_d[0]:
        d.wait()
    gat_d = {0: issue_gat(0)}
    sca_d = {}
    for t in range(NCHUNK):
        if t >= 2:
            for d in sca_d[t - 2]:
                d.wait()
        if t + 2 < NCHUNK:
            idx_d[t + 2] = issue_idx(t + 2)
        if t + 1 < NCHUNK:
            for d in idx_d[t + 1]:
                d.wait()
            gat_d[t + 1] = issue_gat(t + 1)
        for d in gat_d[t]:
            d.wait()
        scale(t)
        sca_d[t] = issue_sca(t)
    for t in range(NCHUNK - 2, NCHUNK):
        for d in sca_d[t]:
            d.wait()

    plsc.subcore_barrier()

    pltpu.sync_copy(acc_sh.at[pl.ds(sid * zrows, zrows)],
                    out_hbm.at[cid, pl.ds(sid * zrows, zrows)])


_sc_call = pl.kernel(
    _sc_body,
    out_type=(jax.ShapeDtypeStruct((NUM_CORES, OUT_LEN, BATCH), jnp.float32),
              jax.ShapeDtypeStruct((NUM_CORES, IN_LEN, BATCH), jnp.float32)),
    mesh=plsc.VectorSubcoreMesh(core_axis_name="c", subcore_axis_name="s"),
    compiler_params=pltpu.CompilerParams(needs_layout_passes=False,
                                         use_tc_tiling_on_sc=False),
    scratch_types=[
        pltpu.VMEM((NBUF, NSEG, SEG), jnp.int32),
        pltpu.VMEM((NBUF, NSEG, SEG), jnp.int32),
        pltpu.VMEM((NBUF, CHUNK), jnp.float32),
        pltpu.VMEM((NBUF, CHUNK, LANES), jnp.float32),
        pltpu.VMEM((BATCH, XPAD), jnp.float32),
        pltpu.VMEM_SHARED((OUT_LEN, BATCH), jnp.float32),
        pltpu.SemaphoreType.DMA((NBUF,)),
        pltpu.SemaphoreType.DMA((NBUF,)),
        pltpu.SemaphoreType.DMA((NBUF,)),
        pltpu.SemaphoreType.DMA,
    ],
)


def _combine_body(parts_ref, bias_ref, out_ref):
    p = (parts_ref[0:OUT_LEN, :] + parts_ref[OUT_LEN:, :]) + bias_ref[...]
    out_ref[...] = p.T


_combine_call = pl.pallas_call(
    _combine_body,
    out_shape=jax.ShapeDtypeStruct((BATCH, OUT_LEN), jnp.float32),
)


def kernel(inputs, kernel, bias, mask_rows, mask_cols):
    x2d = inputs[:, :, 0]
    w = kernel[:, 0]
    rows2d = mask_rows.reshape(NNZ // SEG, SEG)
    cols2d = mask_cols.reshape(NNZ // SEG, SEG)
    parts, _ = _sc_call(x2d, rows2d, cols2d, w)
    out_t = _combine_call(parts.reshape(NUM_CORES * OUT_LEN, BATCH), bias)
    return out_t.reshape(BATCH, OUT_LEN, 1)


# speedup vs baseline: 1.0693x; 1.0054x over previous
import functools

import jax
import jax.numpy as jnp
from jax import lax
from jax.experimental import pallas as pl
from jax.experimental.pallas import tpu as pltpu
from jax.experimental.pallas import tpu_sc as plsc

IN_LEN = 16384
OUT_LEN = 1024
NNZ = 262144
BATCH = 16
LANES = 16

NUM_CORES = 2
NUM_SUBCORES = 16
NW = NUM_CORES * NUM_SUBCORES
PER_W = NNZ // NW
SEG = 128
CHUNK = 1024
NSEG = CHUNK // SEG
NCHUNK = PER_W // CHUNK
GROUPS = CHUNK // LANES
NBUF = 4


XPAD = 1025


def _sc_body(x_hbm, rows_hbm, cols_hbm, w_hbm, out_hbm, xt_s,
             rows_v, cols_v, w_v, gath_v, xb_v, acc_sh,
             sem_idx, sem_gat, sem_sca, sem_tr):
    cid = lax.axis_index("c")
    sid = lax.axis_index("s")
    wid = sid * NUM_CORES + cid
    iota_l = lax.iota(jnp.int32, LANES)

    tcols = IN_LEN // NUM_SUBCORES
    col0 = pl.multiple_of(sid * tcols, tcols)
    tds = [
        pltpu.async_copy(x_hbm.at[b, pl.ds(col0, tcols)],
                         xb_v.at[b, pl.ds(0, tcols)], sem_tr)
        for b in range(BATCH)
    ]
    for d in tds:
        d.wait()
    @plsc.parallel_loop(0, tcols, unroll=4)
    def trow(r):
        gath_v[0, r, :] = plsc.load_gather(
            xb_v, [iota_l, jnp.full((LANES,), r, jnp.int32)])

    pltpu.async_copy(gath_v.at[0], xt_s.at[cid, pl.ds(col0, tcols)],
                     sem_tr).wait()
    xt_hbm = xt_s.at[cid]

    zrows = OUT_LEN // NUM_SUBCORES
    zero = jnp.zeros((LANES,), jnp.float32)

    @plsc.parallel_loop(0, zrows, unroll=4)
    def zb(i):
        gath_v[1, i, :] = zero

    pltpu.sync_copy(gath_v.at[1, pl.ds(0, zrows)],
                    acc_sh.at[pl.ds(sid * zrows, zrows)])
    plsc.subcore_barrier()

    def issue_idx(t):
        b = t % NBUF
        nnz_base = pl.multiple_of(wid * PER_W + t * CHUNK, CHUNK)
        seg_base = pl.multiple_of(nnz_base // SEG, NSEG)
        return [
            pltpu.async_copy(rows_hbm.at[pl.ds(nnz_base, CHUNK)],
                             rows_v.at[b], sem_idx.at[b]),
            pltpu.async_copy(cols_hbm.at[pl.ds(seg_base, NSEG)],
                             cols_v.at[b], sem_idx.at[b]),
            pltpu.async_copy(w_hbm.at[pl.ds(nnz_base, CHUNK)],
                             w_v.at[b], sem_idx.at[b]),
        ]

    def issue_gat(t):
        b = t % NBUF
        return [
            pltpu.async_copy(xt_hbm.at[rows_v.at[b]],
                             gath_v.at[b], sem_gat.at[b])
        ]

    def issue_sca(t):
        b = t % NBUF
        return [
            pltpu.async_copy(gath_v.at[b, pl.ds(s * SEG, SEG)],
                             acc_sh.at[cols_v.at[b, s]], sem_sca.at[b], add=True)
            for s in range(NSEG)
        ]

    def scale(t):
        b = t % NBUF

        @plsc.parallel_loop(0, GROUPS, unroll=4)
        def grp(g):
            gb = g * LANES
            for j in range(LANES):
                pos = gb + j
                wj = plsc.load_gather(
                    w_v.at[b], [jnp.full((LANES,), pos, jnp.int32)])
                gath_v[b, pos, :] = wj * gath_v[b, pos, :]

    idx_d = {0: issue_idx(0), 1: issue_idx(1)}
    for d in idx_d[0]:
        d.wait()
    gat_d = {0: issue_gat(0)}
    sca_d = {}
    for t in range(NCHUNK):
        if t >= 2:
            for d in sca_d[t - 2]:
                d.wait()
        if t + 2 < NCHUNK:
            idx_d[t + 2] = issue_idx(t + 2)
        if t + 1 < NCHUNK:
            for d in idx_d[t + 1]:
                d.wait()
            gat_d[t + 1] = issue_gat(t + 1)
        for d in gat_d[t]:
            d.wait()
        scale(t)
        sca_d[t] = issue_sca(t)
    for t in range(NCHUNK - 2, NCHUNK):
        for d in sca_d[t]:
            d.wait()

    plsc.subcore_barrier()

    pltpu.sync_copy(acc_sh.at[pl.ds(sid * zrows, zrows)],
                    out_hbm.at[cid, pl.ds(sid * zrows, zrows)])


_sc_call = pl.kernel(
    _sc_body,
    out_type=(jax.ShapeDtypeStruct((NUM_CORES, OUT_LEN, BATCH), jnp.float32),
              jax.ShapeDtypeStruct((NUM_CORES, IN_LEN, BATCH), jnp.float32)),
    mesh=plsc.VectorSubcoreMesh(core_axis_name="c", subcore_axis_name="s"),
    compiler_params=pltpu.CompilerParams(needs_layout_passes=False,
                                         use_tc_tiling_on_sc=False),
    scratch_types=[
        pltpu.VMEM((NBUF, CHUNK), jnp.int32),
        pltpu.VMEM((NBUF, NSEG, SEG), jnp.int32),
        pltpu.VMEM((NBUF, CHUNK), jnp.float32),
        pltpu.VMEM((NBUF, CHUNK, LANES), jnp.float32),
        pltpu.VMEM((BATCH, XPAD), jnp.float32),
        pltpu.VMEM_SHARED((OUT_LEN, BATCH), jnp.float32),
        pltpu.SemaphoreType.DMA((NBUF,)),
        pltpu.SemaphoreType.DMA((NBUF,)),
        pltpu.SemaphoreType.DMA((NBUF,)),
        pltpu.SemaphoreType.DMA,
    ],
)


def _combine_body(parts_ref, bias_ref, out_ref):
    p = (parts_ref[0:OUT_LEN, :] + parts_ref[OUT_LEN:, :]) + bias_ref[...]
    out_ref[...] = p.T


_combine_call = pl.pallas_call(
    _combine_body,
    out_shape=jax.ShapeDtypeStruct((BATCH, OUT_LEN), jnp.float32),
)


def kernel(inputs, kernel, bias, mask_rows, mask_cols):
    x2d = inputs[:, :, 0]
    w = kernel[:, 0]
    cols2d = mask_cols.reshape(NNZ // SEG, SEG)
    parts, _ = _sc_call(x2d, mask_rows, cols2d, w)
    out_t = _combine_call(parts.reshape(NUM_CORES * OUT_LEN, BATCH), bias)
    return out_t.reshape(BATCH, OUT_LEN, 1)
